# baseline (speedup 1.0000x reference)
import functools

import jax
import jax.numpy as jnp
from jax import lax
from jax.experimental import pallas as pl
from jax.experimental.pallas import tpu as pltpu
from jax.experimental.pallas import tpu_sc as plsc

B = 4096
L = 200
C = 64
LH = 128
NC = 2
NS = 16
NW = NC * NS
RPT = B // NW
NB = 2
CHUNKS = RPT // NB
NBL = NB * L

_mesh = plsc.VectorSubcoreMesh(core_axis_name="c", subcore_axis_name="s")


@functools.partial(
    pl.kernel,
    out_type=(
        jax.ShapeDtypeStruct((B * C, LH), jnp.float32),
        jax.ShapeDtypeStruct((B * C, LH), jnp.float32),
    ),
    mesh=_mesh,
    scratch_types=[
        pltpu.VMEM((2, NBL), jnp.int32),
        pltpu.VMEM((2, NBL + 16), jnp.float32),
        pltpu.VMEM((2, NBL, C), jnp.float32),
        pltpu.VMEM((2, NB * C, LH), jnp.float32),
        pltpu.VMEM((2, NB * C, LH), jnp.float32),
        pltpu.SemaphoreType.DMA,
        pltpu.SemaphoreType.DMA,
        pltpu.SemaphoreType.DMA,
        pltpu.SemaphoreType.DMA,
    ],
    compiler_params=pltpu.CompilerParams(
        use_tc_tiling_on_sc=False, needs_layout_passes=False
    ),
)
def _emb(table_hbm, x_hbm, mask_hbm, lo_hbm, hi_hbm,
         idx_v, m_v, rows_v, lo_v, hi_v, gsem0, gsem1, ssem0, ssem1):
    gsems = (gsem0, gsem1)
    ssems = (ssem0, ssem1)
    wid = lax.axis_index("s") * NC + lax.axis_index("c")
    iota = lax.iota(jnp.int32, 16)

    def issue(c, p):
        ibase = (wid * RPT + c * NB) * L
        pltpu.sync_copy(x_hbm.at[pl.ds(ibase, NBL)], idx_v.at[p])
        pltpu.sync_copy(mask_hbm.at[pl.ds(ibase, NBL)],
                        m_v.at[p, pl.ds(0, NBL)])
        pltpu.async_copy(table_hbm.at[idx_v.at[p]], rows_v.at[p], gsems[p])
        for j in range(NBL // 16):
            m_v[p, pl.ds(j * 16, 16)] = m_v[p, pl.ds(j * 16, 16)] * 8.0

    def wait_gather(p):
        pltpu.make_async_copy(
            table_hbm.at[idx_v.at[p]], rows_v.at[p], gsems[p]).wait()

    def store(c, p):
        row0 = (wid * RPT + c * NB) * C
        pltpu.async_copy(lo_v.at[p], lo_hbm.at[pl.ds(row0, NB * C)], ssems[p])
        pltpu.async_copy(hi_v.at[p], hi_hbm.at[pl.ds(row0, NB * C)], ssems[p])

    def wait_store(c, p):
        row0 = (wid * RPT + c * NB) * C
        pltpu.make_async_copy(lo_v.at[p], lo_hbm.at[pl.ds(row0, NB * C)],
                              ssems[p]).wait()
        pltpu.make_async_copy(hi_v.at[p], hi_hbm.at[pl.ds(row0, NB * C)],
                              ssems[p]).wait()

    def transpose(p):
        for b in range(NB):
            rowvs = [iota + (b * C + cg * 16) for cg in range(C // 16)]

            @functools.partial(plsc.parallel_loop, 0, LH, unroll=4)
            def _lo_body(l, rowvs=rowvs, b=b):
                r = b * L + l
                mv = m_v[p, pl.ds(r, 16)]
                mm = jnp.full((16,), mv[0], jnp.float32)
                col = jnp.full((16,), l, jnp.int32)
                for cg in range(C // 16):
                    v = rows_v[p, r, pl.ds(cg * 16, 16)]
                    plsc.store_scatter(lo_v.at[p], [rowvs[cg], col], v * mm)

            @functools.partial(plsc.parallel_loop, LH, L, unroll=4)
            def _hi_body(l, rowvs=rowvs, b=b):
                r = b * L + l
                mv = m_v[p, pl.ds(r, 16)]
                mm = jnp.full((16,), mv[0], jnp.float32)
                col = jnp.full((16,), l - LH, jnp.int32)
                for cg in range(C // 16):
                    v = rows_v[p, r, pl.ds(cg * 16, 16)]
                    plsc.store_scatter(hi_v.at[p], [rowvs[cg], col], v * mm)

    issue(0, 0)

    def epoch(e, carry):
        for p in range(2):
            c = 2 * e + p

            @pl.when(c + 1 < CHUNKS)
            def _():
                issue(c + 1, 1 - p)

            wait_gather(p)

            @pl.when(c >= 2)
            def _():
                wait_store(c - 2, p)

            transpose(p)
            store(c, p)
        return carry

    lax.fori_loop(0, CHUNKS // 2, epoch, 0)
    wait_store(CHUNKS - 2, 0)
    wait_store(CHUNKS - 1, 1)


_BB = 8


def _concat_body(lo_ref, hi_ref, out_ref):
    lo = lo_ref[...]
    hi = hi_ref[...]
    for j in range(_BB):
        out_ref[j, :, 0:LH] = lo[j * C:(j + 1) * C, :]
        out_ref[j, :, LH:L] = hi[j * C:(j + 1) * C, 0:L - LH]


_concat = pl.pallas_call(
    _concat_body,
    grid=(B // _BB,),
    in_specs=[
        pl.BlockSpec((_BB * C, LH), lambda i: (i, 0)),
        pl.BlockSpec((_BB * C, LH), lambda i: (i, 0)),
    ],
    out_specs=pl.BlockSpec((_BB, C, L), lambda i: (i, 0, 0)),
    out_shape=jax.ShapeDtypeStruct((B, C, L), jnp.float32),
)


def kernel(x, mask, table):
    x_flat = x.reshape(-1).astype(jnp.int32)
    mask_flat = mask.reshape(-1).astype(jnp.float32)
    lo, hi = _emb(table, x_flat, mask_flat)
    return _concat(lo, hi)


# speedup vs baseline: 1.0604x; 1.0604x over previous
import functools

import jax
import jax.numpy as jnp
from jax import lax
from jax.experimental import pallas as pl
from jax.experimental.pallas import tpu as pltpu
from jax.experimental.pallas import tpu_sc as plsc

B = 4096
L = 200
C = 64
LP = 200
NC = 2
NS = 16
NW = NC * NS
RPT = B // NW
NB = 2
CHUNKS = RPT // NB
NBL = NB * L

_mesh = plsc.VectorSubcoreMesh(core_axis_name="c", subcore_axis_name="s")


@functools.partial(
    pl.kernel,
    out_type=jax.ShapeDtypeStruct((B, C, L), jnp.float32),
    mesh=_mesh,
    scratch_types=[
        pltpu.VMEM((2, NBL), jnp.int32),
        pltpu.VMEM((2, NBL + 16), jnp.float32),
        pltpu.VMEM((2, NBL, C), jnp.float32),
        pltpu.VMEM((2, NB, C, LP), jnp.float32),
        pltpu.SemaphoreType.DMA,
        pltpu.SemaphoreType.DMA,
        pltpu.SemaphoreType.DMA,
        pltpu.SemaphoreType.DMA,
    ],
    compiler_params=pltpu.CompilerParams(
        use_tc_tiling_on_sc=False, needs_layout_passes=False
    ),
)
def _emb(table_hbm, x_hbm, mask_hbm, out_hbm,
         idx_v, m_v, rows_v, out_v, gsem0, gsem1, ssem0, ssem1):
    gsems = (gsem0, gsem1)
    ssems = (ssem0, ssem1)
    wid = lax.axis_index("s") * NC + lax.axis_index("c")
    iota = lax.iota(jnp.int32, 16)

    def issue(c, p):
        ibase = (wid * RPT + c * NB) * L
        pltpu.sync_copy(x_hbm.at[pl.ds(ibase, NBL)], idx_v.at[p])
        pltpu.sync_copy(mask_hbm.at[pl.ds(ibase, NBL)],
                        m_v.at[p, pl.ds(0, NBL)])
        pltpu.async_copy(table_hbm.at[idx_v.at[p]], rows_v.at[p], gsems[p])
        for j in range(NBL // 16):
            m_v[p, pl.ds(j * 16, 16)] = m_v[p, pl.ds(j * 16, 16)] * 8.0

    def wait_gather(p):
        pltpu.make_async_copy(
            table_hbm.at[idx_v.at[p]], rows_v.at[p], gsems[p]).wait()

    def store(c, p):
        b0 = wid * RPT + c * NB
        pltpu.async_copy(out_v.at[p], out_hbm.at[pl.ds(b0, NB)], ssems[p])

    def wait_store(c, p):
        b0 = wid * RPT + c * NB
        pltpu.make_async_copy(out_v.at[p], out_hbm.at[pl.ds(b0, NB)],
                              ssems[p]).wait()

    def transpose(p):
        rowvs = [iota + cg * 16 for cg in range(C // 16)]
        for b in range(NB):

            @functools.partial(plsc.parallel_loop, 0, L, unroll=8)
            def _lbody(l, rowvs=rowvs, b=b):
                r = b * L + l
                mv = m_v[p, pl.ds(r, 16)]
                mm = jnp.full((16,), mv[0], jnp.float32)
                col = jnp.full((16,), l, jnp.int32)
                for cg in range(C // 16):
                    v = rows_v[p, r, pl.ds(cg * 16, 16)]
                    plsc.store_scatter(out_v.at[p, b], [rowvs[cg], col],
                                       v * mm)

    issue(0, 0)

    def epoch(e, carry):
        for p in range(2):
            c = 2 * e + p

            @pl.when(c + 1 < CHUNKS)
            def _():
                issue(c + 1, 1 - p)

            wait_gather(p)

            @pl.when(c >= 2)
            def _():
                wait_store(c - 2, p)

            transpose(p)
            store(c, p)
        return carry

    lax.fori_loop(0, CHUNKS // 2, epoch, 0)
    wait_store(CHUNKS - 2, 0)
    wait_store(CHUNKS - 1, 1)


def kernel(x, mask, table):
    x_flat = x.reshape(-1).astype(jnp.int32)
    mask_flat = mask.reshape(-1).astype(jnp.float32)
    return _emb(table, x_flat, mask_flat)
